# baseline (speedup 1.0000x reference)
import functools

import jax
import jax.numpy as jnp
from jax import lax
from jax.experimental import pallas as pl
from jax.experimental.pallas import tpu as pltpu
from jax.experimental.pallas import tpu_sc as plsc

N = 100000
E = 6400000
NUM_LAYERS = 10

NUM_TILES = 16
NT = 6272
NP = NUM_TILES * NT
VREGS = NT // 16
C = 128
K = 4
EDGES_PER_TILE = 400384
EPAD = NUM_TILES * EDGES_PER_TILE
ROWS_PER_TILE = EDGES_PER_TILE // C
CHUNKS = ROWS_PER_TILE // K


def _rsqrt16(d):
    i = lax.bitcast_convert_type(d, jnp.int32)
    i = jnp.int32(0x5F3759DF) - lax.shift_right_logical(i, 1)
    y = lax.bitcast_convert_type(i, jnp.float32)
    for _ in range(3):
        y = y * (jnp.float32(1.5) - jnp.float32(0.5) * d * y * y)
    return y


def _body(xh, rowr, colr, wv, onesh, out,
          deg, hs0, hs1, t0, t1,
          rowb, colb, rowb2, colb2, m0, m1, n0, n1,
          onesm, disb, tb0, tb1, wbuf,
          sem_l, sem_g, sem_s):
    wid = lax.axis_index("s")
    nbase = wid * NT
    ebase = wid * ROWS_PER_TILE

    pltpu.sync_copy(onesh, onesm)

    def init_deg(i, c):
        pltpu.sync_copy(onesm, deg.at[pl.ds(nbase + i * C, C)])
        return c
    lax.fori_loop(0, NT // C, init_deg, 0)
    if NT % C:
        pltpu.sync_copy(onesm.at[pl.ds(0, NT % C)],
                        deg.at[pl.ds(nbase + (NT // C) * C, NT % C)])
    plsc.subcore_barrier()

    def deg_chunk(c, carry):
        base = ebase + c * K
        pltpu.async_copy(rowr.at[pl.ds(base, K)], rowb, sem_l).wait()
        ss = [pltpu.async_copy(onesm, deg.at[rowb.at[j]], sem_s, add=True)
              for j in range(K)]
        for s_ in ss:
            s_.wait()
        return carry
    lax.fori_loop(0, CHUNKS, deg_chunk, 0)
    plsc.subcore_barrier()

    pltpu.sync_copy(deg.at[pl.ds(nbase, NT)], disb)

    def dis_body(i, c):
        d = disb[pl.ds(i * 16, 16)]
        disb[pl.ds(i * 16, 16)] = _rsqrt16(d)
        return c
    lax.fori_loop(0, VREGS, dis_body, 0)

    def elementwise(l, first):
        if first:
            pltpu.sync_copy(xh.at[0, pl.ds(nbase, NT)], tb0)
            pltpu.sync_copy(xh.at[1, pl.ds(nbase, NT)], tb1)
        else:
            pltpu.sync_copy(t0.at[pl.ds(nbase, NT)], tb0)
            pltpu.sync_copy(t1.at[pl.ds(nbase, NT)], tb1)
        pltpu.sync_copy(wv.at[l], wbuf)

        def ew(i, c):
            s = pl.ds(i * 16, 16)
            dv = disb[s]
            a0 = tb0[s]
            a1 = tb1[s]
            if first:
                h0 = a0
                h1 = a1
            else:
                h0 = dv * a0
                h1 = dv * a1
            u0 = h0 * wbuf[pl.ds(0, 16)] + h1 * wbuf[pl.ds(16, 16)] \
                + wbuf[pl.ds(64, 16)]
            u1 = h0 * wbuf[pl.ds(32, 16)] + h1 * wbuf[pl.ds(48, 16)] \
                + wbuf[pl.ds(80, 16)]
            tb0[s] = dv * u0
            tb1[s] = dv * u1
            return c
        lax.fori_loop(0, VREGS, ew, 0)

        sl = pl.ds(nbase, NT)
        pltpu.sync_copy(tb0, hs0.at[sl])
        pltpu.sync_copy(tb1, hs1.at[sl])
        pltpu.sync_copy(tb0, t0.at[sl])
        pltpu.sync_copy(tb1, t1.at[sl])

    def gat(rb, u0, u1):
        return [pltpu.async_copy(hs0.at[rb.at[j]], u0.at[j], sem_g)
                for j in range(K)] + \
               [pltpu.async_copy(hs1.at[rb.at[j]], u1.at[j], sem_g)
                for j in range(K)]

    def scat(u0, u1, cb):
        ds_ = []
        for j in range(K):
            ds_.append(pltpu.async_copy(u0.at[j], t0.at[cb.at[j]], sem_s,
                                        add=True))
            ds_.append(pltpu.async_copy(u1.at[j], t1.at[cb.at[j]], sem_s,
                                        add=True))
        return ds_

    def edge_pass():
        def chunk(c, carry):
            base = ebase + c * (2 * K)
            la = pltpu.async_copy(rowr.at[pl.ds(base, K)], rowb, sem_l)
            ca = pltpu.async_copy(colr.at[pl.ds(base, K)], colb, sem_l)
            lb = pltpu.async_copy(rowr.at[pl.ds(base + K, K)], rowb2, sem_l)
            cb_ = pltpu.async_copy(colr.at[pl.ds(base + K, K)], colb2, sem_l)
            la.wait()
            ga = gat(rowb, m0, m1)
            ca.wait()
            for g in ga:
                g.wait()
            sa = scat(m0, m1, colb)
            lb.wait()
            gb = gat(rowb2, n0, n1)
            cb_.wait()
            for g in gb:
                g.wait()
            for s_ in sa:
                s_.wait()
            sb = scat(n0, n1, colb2)
            for s_ in sb:
                s_.wait()
            return carry
        lax.fori_loop(0, CHUNKS // 2, chunk, 0)

    elementwise(0, True)
    plsc.subcore_barrier()
    edge_pass()
    plsc.subcore_barrier()

    def layer(l, carry):
        elementwise(l, False)
        plsc.subcore_barrier()
        edge_pass()
        plsc.subcore_barrier()
        return carry
    lax.fori_loop(1, NUM_LAYERS, layer, 0)

    pltpu.sync_copy(t0.at[pl.ds(nbase, NT)], tb0)
    pltpu.sync_copy(t1.at[pl.ds(nbase, NT)], tb1)

    def fin(i, c):
        s = pl.ds(i * 16, 16)
        dv = disb[s]
        tb0[s] = dv * tb0[s]
        tb1[s] = dv * tb1[s]
        return c
    lax.fori_loop(0, VREGS, fin, 0)
    pltpu.sync_copy(tb0, out.at[0, pl.ds(nbase, NT)])
    pltpu.sync_copy(tb1, out.at[1, pl.ds(nbase, NT)])


@jax.jit
def kernel(x, edge, W, b):
    xh = jnp.zeros((2, NP), jnp.float32).at[:, :N].set(x.T)
    epad = EPAD - E
    rowr = jnp.pad(edge[0], (0, epad), constant_values=NP - 1).reshape(
        NUM_TILES * ROWS_PER_TILE, C)
    colr = jnp.pad(edge[1], (0, epad), constant_values=NP - 1).reshape(
        NUM_TILES * ROWS_PER_TILE, C)
    coef = jnp.stack(
        [W[:, 0, 0], W[:, 0, 1], W[:, 1, 0], W[:, 1, 1], b[:, 0], b[:, 1]],
        axis=1)
    wv = jnp.repeat(coef[:, :, None], 16, axis=2).reshape(NUM_LAYERS, 96)
    onesh = jnp.ones((C,), jnp.float32)

    mesh = plsc.VectorSubcoreMesh(core_axis_name="c", subcore_axis_name="s",
                                  num_cores=1)
    run = pl.kernel(
        _body,
        out_type=jax.ShapeDtypeStruct((2, NP), jnp.float32),
        mesh=mesh,
        scratch_types=[
            pltpu.VMEM_SHARED((NP,), jnp.float32),
            pltpu.VMEM_SHARED((NP,), jnp.float32),
            pltpu.VMEM_SHARED((NP,), jnp.float32),
            pltpu.VMEM_SHARED((NP,), jnp.float32),
            pltpu.VMEM_SHARED((NP,), jnp.float32),
            pltpu.VMEM((K, C), jnp.int32),
            pltpu.VMEM((K, C), jnp.int32),
            pltpu.VMEM((K, C), jnp.int32),
            pltpu.VMEM((K, C), jnp.int32),
            pltpu.VMEM((K, C), jnp.float32),
            pltpu.VMEM((K, C), jnp.float32),
            pltpu.VMEM((K, C), jnp.float32),
            pltpu.VMEM((K, C), jnp.float32),
            pltpu.VMEM((C,), jnp.float32),
            pltpu.VMEM((NT,), jnp.float32),
            pltpu.VMEM((NT,), jnp.float32),
            pltpu.VMEM((NT,), jnp.float32),
            pltpu.VMEM((96,), jnp.float32),
            pltpu.SemaphoreType.DMA,
            pltpu.SemaphoreType.DMA,
            pltpu.SemaphoreType.DMA,
        ],
    )
    out = run(xh, rowr, colr, wv, onesh)
    return out[:, :N].T


# speedup vs baseline: 1.4168x; 1.4168x over previous
import functools

import jax
import jax.numpy as jnp
from jax import lax
from jax.experimental import pallas as pl
from jax.experimental.pallas import tpu as pltpu
from jax.experimental.pallas import tpu_sc as plsc

N = 100000
E = 6400000
NUM_LAYERS = 10

NUM_TILES = 16
NT = 6272
NP = NUM_TILES * NT
VREGS = NT // 16
C = 128
K = 8
EDGES_PER_TILE = 400384
EPAD = NUM_TILES * EDGES_PER_TILE
ROWS_PER_TILE = EDGES_PER_TILE // C
CHUNKS = ROWS_PER_TILE // K


def _rsqrt16(d):
    i = lax.bitcast_convert_type(d, jnp.int32)
    i = jnp.int32(0x5F3759DF) - lax.shift_right_logical(i, 1)
    y = lax.bitcast_convert_type(i, jnp.float32)
    for _ in range(3):
        y = y * (jnp.float32(1.5) - jnp.float32(0.5) * d * y * y)
    return y


def _body(xh, rowr, colr, wv, onesh, out,
          deg, hs2, t2,
          rowb, colb, mp, onesm, disb, tb2, wbuf,
          sem_l, sem_g, sem_s):
    wid = lax.axis_index("s")
    nbase = wid * NT
    ebase = wid * ROWS_PER_TILE

    pltpu.sync_copy(onesh, onesm)

    def init_deg(i, c):
        pltpu.sync_copy(onesm, deg.at[pl.ds(nbase + i * C, C)])
        return c
    lax.fori_loop(0, NT // C, init_deg, 0)
    if NT % C:
        pltpu.sync_copy(onesm.at[pl.ds(0, NT % C)],
                        deg.at[pl.ds(nbase + (NT // C) * C, NT % C)])
    plsc.subcore_barrier()

    def deg_chunk(c, carry):
        base = ebase + c * K
        pltpu.async_copy(rowr.at[pl.ds(base, K)], rowb, sem_l).wait()
        ss = [pltpu.async_copy(onesm, deg.at[rowb.at[j]], sem_s, add=True)
              for j in range(K)]
        for s_ in ss:
            s_.wait()
        return carry
    lax.fori_loop(0, CHUNKS, deg_chunk, 0)
    plsc.subcore_barrier()

    pltpu.sync_copy(deg.at[pl.ds(nbase, NT)], disb)

    def dis_body(i, c):
        d = disb[pl.ds(i * 16, 16)]
        disb[pl.ds(i * 16, 16)] = _rsqrt16(d)
        return c
    lax.fori_loop(0, VREGS, dis_body, 0)

    def ew_loop(first):
        def ew(i, c):
            s = pl.ds(i * 16, 16)
            idx = lax.iota(jnp.int32, 16) + i * 16
            f0 = jnp.zeros((16,), jnp.int32)
            f1 = jnp.ones((16,), jnp.int32)
            dv = disb[s]
            a0 = plsc.load_gather(tb2, [idx, f0])
            a1 = plsc.load_gather(tb2, [idx, f1])
            if first:
                h0 = a0
                h1 = a1
            else:
                h0 = dv * a0
                h1 = dv * a1
            u0 = h0 * wbuf[pl.ds(0, 16)] + h1 * wbuf[pl.ds(16, 16)] \
                + wbuf[pl.ds(64, 16)]
            u1 = h0 * wbuf[pl.ds(32, 16)] + h1 * wbuf[pl.ds(48, 16)] \
                + wbuf[pl.ds(80, 16)]
            plsc.store_scatter(tb2, [idx, f0], dv * u0)
            plsc.store_scatter(tb2, [idx, f1], dv * u1)
            return c
        lax.fori_loop(0, VREGS, ew, 0)

    def elementwise(l, first):
        sl = pl.ds(nbase, NT)
        if first:
            pltpu.sync_copy(xh.at[sl], tb2)
        else:
            pltpu.sync_copy(t2.at[sl], tb2)
        pltpu.sync_copy(wv.at[l], wbuf)
        ew_loop(first)
        pltpu.sync_copy(tb2, hs2.at[sl])
        pltpu.sync_copy(tb2, t2.at[sl])

    def edge_pass():
        def chunk(c, carry):
            base = ebase + c * K
            cp1 = pltpu.async_copy(rowr.at[pl.ds(base, K)], rowb, sem_l)
            cp2 = pltpu.async_copy(colr.at[pl.ds(base, K)], colb, sem_l)
            cp1.wait()
            gs = [pltpu.async_copy(hs2.at[rowb.at[j]], mp.at[j], sem_g)
                  for j in range(K)]
            cp2.wait()
            for g in gs:
                g.wait()
            ss = [pltpu.async_copy(mp.at[j], t2.at[colb.at[j]], sem_s,
                                   add=True)
                  for j in range(K)]
            for s_ in ss:
                s_.wait()
            return carry
        lax.fori_loop(0, CHUNKS, chunk, 0)

    elementwise(0, True)
    plsc.subcore_barrier()
    edge_pass()
    plsc.subcore_barrier()

    def layer(l, carry):
        elementwise(l, False)
        plsc.subcore_barrier()
        edge_pass()
        plsc.subcore_barrier()
        return carry
    lax.fori_loop(1, NUM_LAYERS, layer, 0)

    pltpu.sync_copy(t2.at[pl.ds(nbase, NT)], tb2)

    def fin(i, c):
        s = pl.ds(i * 16, 16)
        idx = lax.iota(jnp.int32, 16) + i * 16
        f0 = jnp.zeros((16,), jnp.int32)
        f1 = jnp.ones((16,), jnp.int32)
        dv = disb[s]
        v0 = plsc.load_gather(tb2, [idx, f0])
        v1 = plsc.load_gather(tb2, [idx, f1])
        plsc.store_scatter(tb2, [idx, f0], dv * v0)
        plsc.store_scatter(tb2, [idx, f1], dv * v1)
        return c
    lax.fori_loop(0, VREGS, fin, 0)
    pltpu.sync_copy(tb2, out.at[pl.ds(nbase, NT)])


@jax.jit
def kernel(x, edge, W, b):
    xh = jnp.zeros((NP, 2), jnp.float32).at[:N].set(x)
    epad = EPAD - E
    rowr = jnp.pad(edge[0], (0, epad), constant_values=NP - 1).reshape(
        NUM_TILES * ROWS_PER_TILE, C)
    colr = jnp.pad(edge[1], (0, epad), constant_values=NP - 1).reshape(
        NUM_TILES * ROWS_PER_TILE, C)
    coef = jnp.stack(
        [W[:, 0, 0], W[:, 0, 1], W[:, 1, 0], W[:, 1, 1], b[:, 0], b[:, 1]],
        axis=1)
    wv = jnp.repeat(coef[:, :, None], 16, axis=2).reshape(NUM_LAYERS, 96)
    onesh = jnp.ones((C,), jnp.float32)

    mesh = plsc.VectorSubcoreMesh(core_axis_name="c", subcore_axis_name="s",
                                  num_cores=1)
    run = pl.kernel(
        _body,
        out_type=jax.ShapeDtypeStruct((NP, 2), jnp.float32),
        mesh=mesh,
        compiler_params=pltpu.CompilerParams(
            use_tc_tiling_on_sc=False, needs_layout_passes=False),
        scratch_types=[
            pltpu.VMEM_SHARED((NP,), jnp.float32),
            pltpu.VMEM_SHARED((NP, 2), jnp.float32),
            pltpu.VMEM_SHARED((NP, 2), jnp.float32),
            pltpu.VMEM((K, C), jnp.int32),
            pltpu.VMEM((K, C), jnp.int32),
            pltpu.VMEM((K, C, 2), jnp.float32),
            pltpu.VMEM((C,), jnp.float32),
            pltpu.VMEM((NT,), jnp.float32),
            pltpu.VMEM((NT, 2), jnp.float32),
            pltpu.VMEM((96,), jnp.float32),
            pltpu.SemaphoreType.DMA,
            pltpu.SemaphoreType.DMA,
            pltpu.SemaphoreType.DMA,
        ],
    )
    out = run(xh, rowr, colr, wv, onesh)
    return out[:N]
